# Initial kernel scaffold; baseline (speedup 1.0000x reference)
import jax
import jax.numpy as jnp
from jax.experimental import pallas as pl


def kernel(ft, edge_index_0, edge_index_1, W_proj, W_embed, attn_l, attn_r, Wv, W_tran, b_tran):
    raise NotImplementedError("write your pallas kernel here")


# speedup vs baseline: 8.6291x; 8.6291x over previous
import functools
import jax
import jax.numpy as jnp
from jax import lax
from jax.experimental import pallas as pl
from jax.experimental.pallas import tpu as pltpu
from jax.experimental.pallas import tpu_sc as plsc

N = 10000
NP = 10240
E = 320000
HID = 128
HEADS = 4
NVIEW = 2
NBLK = 2
NTILES = 32
COLS = HID // NTILES
CHUNK = 2000
NCHKS = E // CHUNK
ROWCH = 2560
NGRID = NP // ROWCH


def _projT_body(ft_ref, w_ref, o_ref):
    o_ref[...] = lax.dot_general(
        w_ref[...], ft_ref[...], (((0,), (1,)), ((), ())),
        preferred_element_type=jnp.float32)


def _projT(ftp, w):
    return pl.pallas_call(
        _projT_body,
        grid=(NGRID,),
        in_specs=[pl.BlockSpec((ROWCH, HID), lambda i: (i, 0)),
                  pl.BlockSpec((HID, HID), lambda i: (0, 0))],
        out_specs=pl.BlockSpec((HID, ROWCH), lambda i: (0, i)),
        out_shape=jax.ShapeDtypeStruct((HID, NP), jnp.float32),
    )(ftp, w)


def _zvals_body(c_ref, we_ref, wv_ref, z_ref, vals_ref):
    c = c_ref[...]
    z_ref[...] = lax.dot_general(
        we_ref[...], c, (((0,), (0,)), ((), ())),
        preferred_element_type=jnp.float32)
    for h in range(HEADS):
        vals_ref[h] = lax.dot_general(
            wv_ref[h], c, (((0,), (0,)), ((), ())),
            preferred_element_type=jnp.float32)


def _zvals(cT, we, wv4):
    return pl.pallas_call(
        _zvals_body,
        grid=(NGRID,),
        in_specs=[pl.BlockSpec((HID, ROWCH), lambda i: (0, i)),
                  pl.BlockSpec((HID, HID), lambda i: (0, 0)),
                  pl.BlockSpec((HEADS, HID, HID), lambda i: (0, 0, 0))],
        out_specs=[pl.BlockSpec((HID, ROWCH), lambda i: (0, i)),
                   pl.BlockSpec((HEADS, HID, ROWCH), lambda i: (0, 0, i))],
        out_shape=[jax.ShapeDtypeStruct((HID, NP), jnp.float32),
                   jax.ShapeDtypeStruct((HEADS, HID, NP), jnp.float32)],
    )(cT, we, wv4)


def _att_body(z_ref, al_ref, ar_ref, el_ref, er_ref, m_ref):
    z = z_ref[...]
    el = lax.dot_general(al_ref[...], z, (((1,), (0,)), ((), ())),
                         preferred_element_type=jnp.float32)
    er = lax.dot_general(ar_ref[...], z, (((1,), (0,)), ((), ())),
                         preferred_element_type=jnp.float32)
    el_ref[...] = el
    er_ref[...] = er
    m = jnp.max(el, axis=1) + jnp.max(er, axis=1)
    m = jnp.where(m >= 0, m, 0.2 * m)
    m_ref[...] = jnp.broadcast_to(m[:, None], (HEADS, 16))


def _att(zT, al, ar):
    return pl.pallas_call(
        _att_body,
        out_shape=[jax.ShapeDtypeStruct((HEADS, NP), jnp.float32),
                   jax.ShapeDtypeStruct((HEADS, NP), jnp.float32),
                   jax.ShapeDtypeStruct((HEADS, 16), jnp.float32)],
    )(zT, al, ar)


def _fin_body(num_ref, den_ref, o0_ref, o1_ref):
    for v, o_ref in ((0, o0_ref), (1, o1_ref)):
        acc = jnp.zeros(o_ref.shape, jnp.float32)
        for h in range(HEADS):
            acc = acc + num_ref[v, h] / (den_ref[v, h][None, :] + 1e-9)
        acc = acc * (1.0 / HEADS)
        o_ref[...] = jnp.where(acc >= 0, acc, 0.01 * acc)


def _fin(num, den):
    return pl.pallas_call(
        _fin_body,
        grid=(NGRID,),
        in_specs=[pl.BlockSpec((NVIEW, HEADS, HID, ROWCH),
                               lambda i: (0, 0, 0, i)),
                  pl.BlockSpec((NVIEW, HEADS, ROWCH), lambda i: (0, 0, i))],
        out_specs=[pl.BlockSpec((HID, ROWCH), lambda i: (0, i)),
                   pl.BlockSpec((HID, ROWCH), lambda i: (0, i))],
        out_shape=[jax.ShapeDtypeStruct((HID, NP), jnp.float32),
                   jax.ShapeDtypeStruct((HID, NP), jnp.float32)],
    )(num, den)


def _final_body(p_ref, c0_ref, c1_ref, w0_ref, w1_ref, w2_ref, b_ref, o_ref):
    r = lax.dot_general(p_ref[...], w0_ref[...], (((0,), (0,)), ((), ())),
                        preferred_element_type=jnp.float32)
    r = r + lax.dot_general(c0_ref[...], w1_ref[...], (((0,), (0,)), ((), ())),
                            preferred_element_type=jnp.float32)
    r = r + lax.dot_general(c1_ref[...], w2_ref[...], (((0,), (0,)), ((), ())),
                            preferred_element_type=jnp.float32)
    o_ref[...] = r + b_ref[...]


def _final(projT, c0T, c1T, w0, w1, w2, b2d):
    return pl.pallas_call(
        _final_body,
        grid=(NGRID,),
        in_specs=[pl.BlockSpec((HID, ROWCH), lambda i: (0, i)),
                  pl.BlockSpec((HID, ROWCH), lambda i: (0, i)),
                  pl.BlockSpec((HID, ROWCH), lambda i: (0, i)),
                  pl.BlockSpec((HID, HID), lambda i: (0, 0)),
                  pl.BlockSpec((HID, HID), lambda i: (0, 0)),
                  pl.BlockSpec((HID, HID), lambda i: (0, 0)),
                  pl.BlockSpec((1, HID), lambda i: (0, 0))],
        out_specs=pl.BlockSpec((ROWCH, HID), lambda i: (i, 0)),
        out_shape=jax.ShapeDtypeStruct((NP, HID), jnp.float32),
    )(projT, c0T, c1T, w0, w1, w2, b2d)


_sc_mesh = plsc.VectorSubcoreMesh(core_axis_name="c", subcore_axis_name="s")


@functools.partial(
    pl.kernel,
    mesh=_sc_mesh,
    compiler_params=pltpu.CompilerParams(needs_layout_passes=False),
    out_type=[jax.ShapeDtypeStruct((NVIEW * HEADS * HID * NP,), jnp.float32),
              jax.ShapeDtypeStruct((NVIEW * HEADS * NP,), jnp.float32)],
    scratch_types=[pltpu.VMEM((NP,), jnp.float32),
                   pltpu.VMEM((NP,), jnp.float32),
                   pltpu.VMEM((COLS * NP,), jnp.float32),
                   pltpu.VMEM((COLS * NP,), jnp.float32),
                   pltpu.VMEM((NP,), jnp.float32),
                   pltpu.VMEM((16,), jnp.float32),
                   pltpu.VMEM((CHUNK,), jnp.int32),
                   pltpu.VMEM((CHUNK,), jnp.int32)],
)
def _sc_edge(elT, erT, valsT, msh, src2, dst2, num_out, den_out,
             el_v, er_v, vals_v, num_v, den_v, m_v, sidx_v, didx_v):
    wid = lax.axis_index("s") * 2 + lax.axis_index("c")
    cbase = wid * COLS
    for v in range(NVIEW):
        for h in range(HEADS):
            vh = v * HEADS + h
            pltpu.sync_copy(elT.at[pl.ds(vh * NP, NP)], el_v)
            pltpu.sync_copy(erT.at[pl.ds(vh * NP, NP)], er_v)
            pltpu.sync_copy(
                valsT.at[pl.ds((vh * HID + cbase) * NP, COLS * NP)], vals_v)
            pltpu.sync_copy(msh.at[pl.ds(vh * 16, 16)], m_v)
            mval = m_v[...]

            def zbody(i, _):
                z16 = jnp.zeros((16,), jnp.float32)
                for c in range(COLS):
                    num_v[pl.ds(c * NP + i * 16, 16)] = z16
                den_v[pl.ds(i * 16, 16)] = z16
                return 0
            lax.fori_loop(0, NP // 16, zbody, 0)

            def cbody(k, _):
                off = v * E + k * CHUNK
                pltpu.sync_copy(src2.at[pl.ds(off, CHUNK)], sidx_v)
                pltpu.sync_copy(dst2.at[pl.ds(off, CHUNK)], didx_v)

                def gbody(g, _):
                    s = sidx_v[pl.ds(g * 16, 16)]
                    d = didx_v[pl.ds(g * 16, 16)]
                    a = plsc.load_gather(el_v, [s])
                    b = plsc.load_gather(er_v, [d])
                    l = a + b
                    l = jnp.where(l >= 0, l, l * 0.2)
                    e = jnp.exp(l - mval)
                    for c in range(COLS):
                        gv = plsc.load_gather(vals_v, [s + c * NP])
                        plsc.addupdate_scatter(num_v, [d + c * NP], gv * e)
                    plsc.addupdate_scatter(den_v, [d], e)
                    return 0
                lax.fori_loop(0, CHUNK // 16, gbody, 0)
                return 0
            lax.fori_loop(0, NCHKS, cbody, 0)

            pltpu.sync_copy(
                num_v, num_out.at[pl.ds((vh * HID + cbase) * NP, COLS * NP)])

            @pl.when(wid == 0)
            def _():
                pltpu.sync_copy(den_v, den_out.at[pl.ds(vh * NP, NP)])


def kernel(ft, edge_index_0, edge_index_1, W_proj, W_embed, attn_l, attn_r,
           Wv, W_tran, b_tran):
    ftp = jnp.pad(ft, ((0, NP - N), (0, 0)))
    src2 = jnp.concatenate([edge_index_0[0], edge_index_1[0]])
    dst2 = jnp.concatenate([edge_index_0[1], edge_index_1[1]])

    projT = _projT(ftp, W_proj)
    cT = [projT, projT]
    for i in range(NBLK):
        els, ers, mshs, valss = [], [], [], []
        for j in range(NVIEW):
            zT, valsT = _zvals(cT[j], W_embed, Wv[j, i])
            elT, erT, mT = _att(zT, attn_l[j, i], attn_r[j, i])
            els.append(elT)
            ers.append(erT)
            mshs.append(mT)
            valss.append(valsT)
        elS = jnp.stack(els).reshape(-1)
        erS = jnp.stack(ers).reshape(-1)
        mS = jnp.stack(mshs).reshape(-1)
        valsS = jnp.stack(valss).reshape(-1)
        num, den = _sc_edge(elS, erS, valsS, mS, src2, dst2)
        cT = list(_fin(num.reshape(NVIEW, HEADS, HID, NP),
                       den.reshape(NVIEW, HEADS, NP)))

    out = _final(projT, cT[0], cT[1],
                 W_tran[0:HID], W_tran[HID:2 * HID], W_tran[2 * HID:],
                 b_tran.reshape(1, HID))
    return out[:N]
